# Initial kernel scaffold; baseline (speedup 1.0000x reference)
import jax
import jax.numpy as jnp
from jax.experimental import pallas as pl


def kernel(x, edge_index, edge_weight, W1, b1, W2, b2):
    raise NotImplementedError("write your pallas kernel here")


# speedup vs baseline: 6.9946x; 6.9946x over previous
import jax
import jax.numpy as jnp
from jax import lax
from jax.experimental import pallas as pl
from jax.experimental.pallas import tpu as pltpu
from jax.experimental.pallas import tpu_sc as plsc

_N = 10000
_E = 320000
_D = 128
_H = 64
_C = 64
_ALPHA = 0.1
_K = 10

_NC = 2
_NS = 16
_NW = _NC * _NS
_EPT = _E // _NW
_G = 80
_NCHUNK = _EPT // _G
_NP = 10240
_RPT = _NP // _NS

_ROWBLK = 1000


def _mlp_block(x_ref, w1_ref, b1_ref, w2_ref, b2_ref, z_ref):
    h = jnp.dot(x_ref[...], w1_ref[...], preferred_element_type=jnp.float32)
    h = jnp.maximum(h + b1_ref[...], 0.0)
    z_ref[...] = jnp.dot(h, w2_ref[...], preferred_element_type=jnp.float32) + b2_ref[...]


def _combine_block(a0_ref, a1_ref, z_ref, o_ref):
    o_ref[...] = (1.0 - _ALPHA) * (a0_ref[0] + a1_ref[0]) + _ALPHA * z_ref[...]


def _final_block(a0_ref, a1_ref, z_ref, o_ref):
    xk = (1.0 - _ALPHA) * (a0_ref[0] + a1_ref[0]) + _ALPHA * z_ref[...]
    m = jnp.max(xk, axis=1, keepdims=True)
    s = jnp.sum(jnp.exp(xk - m), axis=1, keepdims=True)
    o_ref[...] = (xk - m) - jnp.log(s)


def _prop_step_body(xk_hbm, src_hbm, dst_hbm, w_hbm, zeros_hbm, part_hbm,
                    acc, src_v, dst_v, w_v, rows_v, sem):
    cid = lax.axis_index("c")
    sid = lax.axis_index("s")
    gid = cid * _NS + sid

    pltpu.sync_copy(zeros_hbm, acc.at[pl.ds(sid * _RPT, _RPT)])
    pltpu.sync_copy(src_hbm.at[gid], src_v)
    pltpu.sync_copy(dst_hbm.at[gid], dst_v)
    pltpu.sync_copy(w_hbm.at[gid], w_v)
    plsc.subcore_barrier()

    def chunk(i, carry):
        pltpu.async_copy(xk_hbm.at[src_v.at[i]], rows_v, sem).wait()
        for r in range(_G // 16):
            w16 = w_v[i, pl.ds(r * 16, 16)]
            for e in range(16):
                wv = w16[e]
                row = r * 16 + e
                for cpart in range(_C // 16):
                    sl = pl.ds(cpart * 16, 16)
                    rows_v[row, sl] = rows_v[row, sl] * wv
        pltpu.sync_copy(rows_v, acc.at[dst_v.at[i]], add=True)
        return carry

    lax.fori_loop(0, _NCHUNK, chunk, 0)

    plsc.subcore_barrier()
    pltpu.sync_copy(acc.at[pl.ds(sid * _RPT, _RPT)],
                    part_hbm.at[cid, pl.ds(sid * _RPT, _RPT)])


def _make_prop_step():
    mesh = plsc.VectorSubcoreMesh(core_axis_name="c", subcore_axis_name="s")
    return pl.kernel(
        _prop_step_body,
        out_type=jax.ShapeDtypeStruct((_NC, _NP, _C), jnp.float32),
        mesh=mesh,
        scratch_types=[
            pltpu.VMEM_SHARED((_NP, _C), jnp.float32),
            pltpu.VMEM((_NCHUNK, _G), jnp.int32),
            pltpu.VMEM((_NCHUNK, _G), jnp.int32),
            pltpu.VMEM((_NCHUNK, _G), jnp.float32),
            pltpu.VMEM((_G, _C), jnp.float32),
            pltpu.SemaphoreType.DMA,
        ],
        compiler_params=pltpu.CompilerParams(use_tc_tiling_on_sc=False),
    )


def _dense(body, n_in3, extra_in=0):
    in_specs = [
        pl.BlockSpec((1, _ROWBLK, _C), lambda i: (0, i, 0)),
        pl.BlockSpec((1, _ROWBLK, _C), lambda i: (1, i, 0)),
        pl.BlockSpec((_ROWBLK, _C), lambda i: (i, 0)),
    ]
    return pl.pallas_call(
        body,
        grid=(_N // _ROWBLK,),
        in_specs=in_specs,
        out_specs=pl.BlockSpec((_ROWBLK, _C), lambda i: (i, 0)),
        out_shape=jax.ShapeDtypeStruct((_N, _C), jnp.float32),
    )


def kernel(x, edge_index, edge_weight, W1, b1, W2, b2):
    dst = edge_index[0].reshape(_NW, _NCHUNK, _G)
    src = edge_index[1].reshape(_NW, _NCHUNK, _G)
    w = edge_weight.reshape(_NW, _NCHUNK, _G)
    zeros = jnp.zeros((_RPT, _C), jnp.float32)

    mlp = pl.pallas_call(
        _mlp_block,
        grid=(_N // _ROWBLK,),
        in_specs=[
            pl.BlockSpec((_ROWBLK, _D), lambda i: (i, 0)),
            pl.BlockSpec((_D, _H), lambda i: (0, 0)),
            pl.BlockSpec((1, _H), lambda i: (0, 0)),
            pl.BlockSpec((_H, _C), lambda i: (0, 0)),
            pl.BlockSpec((1, _C), lambda i: (0, 0)),
        ],
        out_specs=pl.BlockSpec((_ROWBLK, _C), lambda i: (i, 0)),
        out_shape=jax.ShapeDtypeStruct((_N, _C), jnp.float32),
    )
    z = mlp(x, W1, b1.reshape(1, _H), W2, b2.reshape(1, _C))

    step = _make_prop_step()
    combine = _dense(_combine_block, 3)
    final = _dense(_final_block, 3)

    xk = z
    for k in range(_K):
        part = step(xk, src, dst, w, zeros)
        blend = final if k == _K - 1 else combine
        xk = blend(part, part, z)
    return xk


# speedup vs baseline: 9.6104x; 1.3740x over previous
import jax
import jax.numpy as jnp
from jax import lax
from jax.experimental import pallas as pl
from jax.experimental.pallas import tpu as pltpu
from jax.experimental.pallas import tpu_sc as plsc

_N = 10000
_E = 320000
_D = 128
_H = 64
_C = 64
_ALPHA = 0.1
_K = 10

_NC = 2
_NS = 16
_NW = _NC * _NS
_EPT = _E // _NW
_G = 80
_NCHUNK = _EPT // _G
_NP = 10240
_RPT = _NP // _NS

_ROWBLK = 1000


def _mlp_block(x_ref, w1_ref, b1_ref, w2_ref, b2_ref, z_ref):
    h = jnp.dot(x_ref[...], w1_ref[...], preferred_element_type=jnp.float32)
    h = jnp.maximum(h + b1_ref[...], 0.0)
    z_ref[...] = jnp.dot(h, w2_ref[...], preferred_element_type=jnp.float32) + b2_ref[...]


def _combine_block(a0_ref, a1_ref, z_ref, o_ref):
    o_ref[...] = (1.0 - _ALPHA) * (a0_ref[0] + a1_ref[0]) + _ALPHA * z_ref[...]


def _final_block(a0_ref, a1_ref, z_ref, o_ref):
    xk = (1.0 - _ALPHA) * (a0_ref[0] + a1_ref[0]) + _ALPHA * z_ref[...]
    m = jnp.max(xk, axis=1, keepdims=True)
    s = jnp.sum(jnp.exp(xk - m), axis=1, keepdims=True)
    o_ref[...] = (xk - m) - jnp.log(s)


def _prop_step_body(xk_hbm, src_hbm, dst_hbm, w_hbm, zeros_hbm, part_hbm,
                    acc, src_v, dst_v, w_v, rows0, rows1, gsem0, gsem1):
    cid = lax.axis_index("c")
    sid = lax.axis_index("s")
    gid = cid * _NS + sid

    pltpu.sync_copy(zeros_hbm, acc.at[pl.ds(sid * _RPT, _RPT)])
    pltpu.sync_copy(src_hbm.at[gid], src_v)
    pltpu.sync_copy(dst_hbm.at[gid], dst_v)
    pltpu.sync_copy(w_hbm.at[gid], w_v)
    plsc.subcore_barrier()

    rows = (rows0, rows1)
    gsems = (gsem0, gsem1)

    def scale(buf, j):
        for r in range(_G // 16):
            w16 = w_v[j, pl.ds(r * 16, 16)]
            for e in range(16):
                wv = w16[e]
                row = r * 16 + e
                for cpart in range(_C // 16):
                    sl = pl.ds(cpart * 16, 16)
                    buf[row, sl] = buf[row, sl] * wv

    pltpu.async_copy(xk_hbm.at[src_v.at[0]], rows0, gsem0)

    def body(t, carry):
        for p in range(2):
            i = 2 * t + p
            buf, gsem = rows[p], gsems[p]
            nbuf, ngsem = rows[1 - p], gsems[1 - p]

            @pl.when(i < _NCHUNK)
            def _():
                pltpu.make_async_copy(xk_hbm.at[src_v.at[i]], buf, gsem).wait()

                @pl.when(i < _NCHUNK - 1)
                def _():
                    pltpu.async_copy(xk_hbm.at[src_v.at[i + 1]], nbuf, ngsem)

                scale(buf, i)
                pltpu.sync_copy(buf, acc.at[dst_v.at[i]], add=True)
        return carry

    lax.fori_loop(0, (_NCHUNK + 1) // 2, body, 0)

    plsc.subcore_barrier()
    pltpu.sync_copy(acc.at[pl.ds(sid * _RPT, _RPT)],
                    part_hbm.at[cid, pl.ds(sid * _RPT, _RPT)])


def _make_prop_step():
    mesh = plsc.VectorSubcoreMesh(core_axis_name="c", subcore_axis_name="s")
    return pl.kernel(
        _prop_step_body,
        out_type=jax.ShapeDtypeStruct((_NC, _NP, _C), jnp.float32),
        mesh=mesh,
        scratch_types=[
            pltpu.VMEM_SHARED((_NP, _C), jnp.float32),
            pltpu.VMEM((_NCHUNK, _G), jnp.int32),
            pltpu.VMEM((_NCHUNK, _G), jnp.int32),
            pltpu.VMEM((_NCHUNK, _G), jnp.float32),
            pltpu.VMEM((_G, _C), jnp.float32),
            pltpu.VMEM((_G, _C), jnp.float32),
            pltpu.SemaphoreType.DMA,
            pltpu.SemaphoreType.DMA,
        ],
        compiler_params=pltpu.CompilerParams(use_tc_tiling_on_sc=False),
    )


def _dense(body, n_in3, extra_in=0):
    in_specs = [
        pl.BlockSpec((1, _ROWBLK, _C), lambda i: (0, i, 0)),
        pl.BlockSpec((1, _ROWBLK, _C), lambda i: (1, i, 0)),
        pl.BlockSpec((_ROWBLK, _C), lambda i: (i, 0)),
    ]
    return pl.pallas_call(
        body,
        grid=(_N // _ROWBLK,),
        in_specs=in_specs,
        out_specs=pl.BlockSpec((_ROWBLK, _C), lambda i: (i, 0)),
        out_shape=jax.ShapeDtypeStruct((_N, _C), jnp.float32),
    )


def kernel(x, edge_index, edge_weight, W1, b1, W2, b2):
    dst = edge_index[0].reshape(_NW, _NCHUNK, _G)
    src = edge_index[1].reshape(_NW, _NCHUNK, _G)
    w = edge_weight.reshape(_NW, _NCHUNK, _G)
    zeros = jnp.zeros((_RPT, _C), jnp.float32)

    mlp = pl.pallas_call(
        _mlp_block,
        grid=(_N // _ROWBLK,),
        in_specs=[
            pl.BlockSpec((_ROWBLK, _D), lambda i: (i, 0)),
            pl.BlockSpec((_D, _H), lambda i: (0, 0)),
            pl.BlockSpec((1, _H), lambda i: (0, 0)),
            pl.BlockSpec((_H, _C), lambda i: (0, 0)),
            pl.BlockSpec((1, _C), lambda i: (0, 0)),
        ],
        out_specs=pl.BlockSpec((_ROWBLK, _C), lambda i: (i, 0)),
        out_shape=jax.ShapeDtypeStruct((_N, _C), jnp.float32),
    )
    z = mlp(x, W1, b1.reshape(1, _H), W2, b2.reshape(1, _C))

    step = _make_prop_step()
    combine = _dense(_combine_block, 3)
    final = _dense(_final_block, 3)

    xk = z
    for k in range(_K):
        part = step(xk, src, dst, w, zeros)
        blend = final if k == _K - 1 else combine
        xk = blend(part, part, z)
    return xk
